# Initial kernel scaffold; baseline (speedup 1.0000x reference)
import functools

import jax
import jax.numpy as jnp
from jax import lax
from jax.experimental import pallas as pl
from jax.experimental.pallas import tpu as pltpu
from jax.experimental.pallas import tpu_sc as plsc

N = 10000
E = 320000
D = 128
DEG_W = 16
CHUNK = 80


def _edge_body(nc, ns, hs, srcg, dstg, zeros, out, sidx, didx, rows, acc, sem):
    c = lax.axis_index("c")
    s = lax.axis_index("s")
    wid = s * nc + c
    nw = nc * ns
    rows_per_s = N // ns
    per_tile = E // nw
    nchunk = per_tile // CHUNK

    pltpu.sync_copy(zeros.at[pl.ds(s * rows_per_s, rows_per_s)],
                    acc.at[pl.ds(s * rows_per_s, rows_per_s)])
    plsc.subcore_barrier()

    base = wid * per_tile

    def chunk(i, carry):
        off = base + i * CHUNK
        pltpu.sync_copy(srcg.at[pl.ds(off, CHUNK)], sidx)
        pltpu.sync_copy(dstg.at[pl.ds(off, CHUNK)], didx)
        pltpu.async_copy(hs.at[sidx], rows, sem).wait()
        pltpu.sync_copy(rows, acc.at[didx], add=True)
        return carry

    lax.fori_loop(0, nchunk, chunk, 0)
    plsc.subcore_barrier()
    pltpu.sync_copy(acc.at[pl.ds(s * rows_per_s, rows_per_s)],
                    out.at[c].at[pl.ds(s * rows_per_s, rows_per_s)])


def _deg_body(nc, ns, dstg, ones, zeros, out, didx, ones_v, acc, sem):
    c = lax.axis_index("c")
    s = lax.axis_index("s")
    wid = s * nc + c
    nw = nc * ns
    rows_per_s = N // ns
    per_tile = E // nw
    nchunk = per_tile // CHUNK

    pltpu.sync_copy(ones, ones_v)
    pltpu.sync_copy(zeros.at[pl.ds(s * rows_per_s, rows_per_s)],
                    acc.at[pl.ds(s * rows_per_s, rows_per_s)])
    plsc.subcore_barrier()

    base = wid * per_tile

    def chunk(i, carry):
        off = base + i * CHUNK
        pltpu.sync_copy(dstg.at[pl.ds(off, CHUNK)], didx)
        pltpu.sync_copy(ones_v, acc.at[didx], add=True)
        return carry

    lax.fori_loop(0, nchunk, chunk, 0)
    plsc.subcore_barrier()
    pltpu.sync_copy(acc.at[pl.ds(s * rows_per_s, rows_per_s)],
                    out.at[c].at[pl.ds(s * rows_per_s, rows_per_s)])


def _make_sc_kernels():
    info = plsc.get_sparse_core_info()
    nc, ns = info.num_cores, info.num_subcores
    mesh = plsc.VectorSubcoreMesh(core_axis_name="c", subcore_axis_name="s")

    edge = pl.kernel(
        functools.partial(_edge_body, nc, ns),
        out_type=jax.ShapeDtypeStruct((nc, N, D), jnp.float32),
        mesh=mesh,
        scratch_types=[
            pltpu.VMEM((CHUNK,), jnp.int32),
            pltpu.VMEM((CHUNK,), jnp.int32),
            pltpu.VMEM((CHUNK, D), jnp.float32),
            pltpu.VMEM_SHARED((N, D), jnp.float32),
            pltpu.SemaphoreType.DMA,
        ],
    )

    deg = pl.kernel(
        functools.partial(_deg_body, nc, ns),
        out_type=jax.ShapeDtypeStruct((nc, N, DEG_W), jnp.float32),
        mesh=mesh,
        scratch_types=[
            pltpu.VMEM((CHUNK,), jnp.int32),
            pltpu.VMEM((CHUNK, DEG_W), jnp.float32),
            pltpu.VMEM_SHARED((N, DEG_W), jnp.float32),
            pltpu.SemaphoreType.DMA,
        ],
    )
    return edge, deg


def _tc_first_body(degp_ref, x_ref, w_ref, dinv_ref, hs_ref):
    degp = degp_ref[...]
    deg = degp[0, :, 0:1] + degp[1, :, 0:1] + 1.0
    dinv = lax.rsqrt(deg)
    dinv_ref[...] = dinv
    hs_ref[...] = dinv * jnp.dot(x_ref[...], w_ref[...],
                                 preferred_element_type=jnp.float32)


def _tc_mid_body(p_ref, hs_ref, dinv_ref, b_ref, g_ref, beta_ref, w_ref, o_ref):
    p = p_ref[...]
    dinv = dinv_ref[...]
    t = dinv * (p[0] + p[1] + hs_ref[...]) + b_ref[...]
    mean = jnp.mean(t, axis=0, keepdims=True)
    var = jnp.mean((t - mean) * (t - mean), axis=0, keepdims=True)
    y = (t - mean) * lax.rsqrt(var + 1e-5) * g_ref[...] + beta_ref[...]
    r = jnp.maximum(y, 0.0)
    o_ref[...] = dinv * jnp.dot(r, w_ref[...],
                                preferred_element_type=jnp.float32)


def _tc_final_body(p_ref, hs_ref, dinv_ref, b_ref, o_ref):
    p = p_ref[...]
    o_ref[...] = dinv_ref[...] * (p[0] + p[1] + hs_ref[...]) + b_ref[...]


_tc_first = pl.pallas_call(
    _tc_first_body,
    out_shape=(jax.ShapeDtypeStruct((N, 1), jnp.float32),
               jax.ShapeDtypeStruct((N, D), jnp.float32)),
)

_tc_mid = pl.pallas_call(
    _tc_mid_body,
    out_shape=jax.ShapeDtypeStruct((N, D), jnp.float32),
)

_tc_final = pl.pallas_call(
    _tc_final_body,
    out_shape=jax.ShapeDtypeStruct((N, D), jnp.float32),
)


def kernel(x, edge_index, W1, b1, W2, b2, W3, b3, W4, b4,
           g1, beta1, g2, beta2, g3, beta3):
    edge = jnp.asarray(edge_index, jnp.int32)
    src = edge[0]
    dst = edge[1]

    edge_call, deg_call = _make_sc_kernels()

    zeros_nd = jnp.zeros((N, D), jnp.float32)
    zeros_nw = jnp.zeros((N, DEG_W), jnp.float32)
    ones_cw = jnp.ones((CHUNK, DEG_W), jnp.float32)

    degp = deg_call(dst, ones_cw, zeros_nw)
    dinv, hs = _tc_first(degp, x, W1)

    b1r, b2r, b3r, b4r = (v.reshape(1, D) for v in (b1, b2, b3, b4))
    g1r, g2r, g3r = (v.reshape(1, D) for v in (g1, g2, g3))
    be1, be2, be3 = (v.reshape(1, D) for v in (beta1, beta2, beta3))

    p = edge_call(hs, src, dst, zeros_nd)
    hs = _tc_mid(p, hs, dinv, b1r, g1r, be1, W2)
    p = edge_call(hs, src, dst, zeros_nd)
    hs = _tc_mid(p, hs, dinv, b2r, g2r, be2, W3)
    p = edge_call(hs, src, dst, zeros_nd)
    hs = _tc_mid(p, hs, dinv, b3r, g3r, be3, W4)
    p = edge_call(hs, src, dst, zeros_nd)
    return _tc_final(p, hs, dinv, b4r)


# speedup vs baseline: 9.8634x; 9.8634x over previous
import functools

import jax
import jax.numpy as jnp
from jax import lax
from jax.experimental import pallas as pl
from jax.experimental.pallas import tpu as pltpu
from jax.experimental.pallas import tpu_sc as plsc

N = 10000
E = 320000
D = 128
DEG_W = 128
CHUNK = 80


def _slice_copy(src, dst, s, ns):
    rows_per_s = (N // ns // 8) * 8
    tail = N - ns * rows_per_s
    pltpu.sync_copy(src.at[pl.ds(s * rows_per_s, rows_per_s)],
                    dst.at[pl.ds(s * rows_per_s, rows_per_s)])
    if tail:
        @pl.when(s == 0)
        def _():
            pltpu.sync_copy(src.at[pl.ds(ns * rows_per_s, tail)],
                            dst.at[pl.ds(ns * rows_per_s, tail)])


def _edge_body(nc, ns, hs, srcg, dstg, zeros, out, sidx, didx, rows, acc, sem):
    c = lax.axis_index("c")
    s = lax.axis_index("s")
    wid = s * nc + c
    nw = nc * ns
    per_tile = E // nw
    nchunk = per_tile // CHUNK

    _slice_copy(zeros, acc, s, ns)
    plsc.subcore_barrier()

    base = wid * per_tile

    def chunk(i, carry):
        off = base + i * CHUNK
        pltpu.sync_copy(srcg.at[pl.ds(off, CHUNK)], sidx)
        pltpu.sync_copy(dstg.at[pl.ds(off, CHUNK)], didx)
        pltpu.async_copy(hs.at[sidx], rows, sem).wait()
        pltpu.sync_copy(rows, acc.at[didx], add=True)
        return carry

    lax.fori_loop(0, nchunk, chunk, 0)
    plsc.subcore_barrier()
    _slice_copy(acc, out.at[c], s, ns)


def _deg_body(nc, ns, dstg, ones, zeros, out, didx, ones_v, acc, sem):
    c = lax.axis_index("c")
    s = lax.axis_index("s")
    wid = s * nc + c
    nw = nc * ns
    per_tile = E // nw
    nchunk = per_tile // CHUNK

    pltpu.sync_copy(ones, ones_v)
    _slice_copy(zeros, acc, s, ns)
    plsc.subcore_barrier()

    base = wid * per_tile

    def chunk(i, carry):
        off = base + i * CHUNK
        pltpu.sync_copy(dstg.at[pl.ds(off, CHUNK)], didx)
        pltpu.sync_copy(ones_v, acc.at[didx], add=True)
        return carry

    lax.fori_loop(0, nchunk, chunk, 0)
    plsc.subcore_barrier()
    _slice_copy(acc, out.at[c], s, ns)


def _make_sc_kernels():
    info = plsc.get_sparse_core_info()
    nc, ns = info.num_cores, info.num_subcores
    mesh = plsc.VectorSubcoreMesh(core_axis_name="c", subcore_axis_name="s")

    edge = pl.kernel(
        functools.partial(_edge_body, nc, ns),
        out_type=jax.ShapeDtypeStruct((nc, N, D), jnp.float32),
        mesh=mesh,
        scratch_types=[
            pltpu.VMEM((CHUNK,), jnp.int32),
            pltpu.VMEM((CHUNK,), jnp.int32),
            pltpu.VMEM((CHUNK, D), jnp.float32),
            pltpu.VMEM_SHARED((N, D), jnp.float32),
            pltpu.SemaphoreType.DMA,
        ],
    )

    deg = pl.kernel(
        functools.partial(_deg_body, nc, ns),
        out_type=jax.ShapeDtypeStruct((nc, N, DEG_W), jnp.float32),
        mesh=mesh,
        scratch_types=[
            pltpu.VMEM((CHUNK,), jnp.int32),
            pltpu.VMEM((CHUNK, DEG_W), jnp.float32),
            pltpu.VMEM_SHARED((N, DEG_W), jnp.float32),
            pltpu.SemaphoreType.DMA,
        ],
    )
    return edge, deg


def _tc_first_body(degp_ref, x_ref, w_ref, dinv_ref, hs_ref):
    degp = degp_ref[...]
    deg = degp[0, :, 0:1] + degp[1, :, 0:1] + 1.0
    dinv = lax.rsqrt(deg)
    dinv_ref[...] = dinv
    hs_ref[...] = dinv * jnp.dot(x_ref[...], w_ref[...],
                                 preferred_element_type=jnp.float32)


def _tc_mid_body(p_ref, hs_ref, dinv_ref, b_ref, g_ref, beta_ref, w_ref, o_ref):
    p = p_ref[...]
    dinv = dinv_ref[...]
    t = dinv * (p[0] + p[1] + hs_ref[...]) + b_ref[...]
    mean = jnp.mean(t, axis=0, keepdims=True)
    var = jnp.mean((t - mean) * (t - mean), axis=0, keepdims=True)
    y = (t - mean) * lax.rsqrt(var + 1e-5) * g_ref[...] + beta_ref[...]
    r = jnp.maximum(y, 0.0)
    o_ref[...] = dinv * jnp.dot(r, w_ref[...],
                                preferred_element_type=jnp.float32)


def _tc_final_body(p_ref, hs_ref, dinv_ref, b_ref, o_ref):
    p = p_ref[...]
    o_ref[...] = dinv_ref[...] * (p[0] + p[1] + hs_ref[...]) + b_ref[...]


_tc_first = pl.pallas_call(
    _tc_first_body,
    out_shape=(jax.ShapeDtypeStruct((N, 1), jnp.float32),
               jax.ShapeDtypeStruct((N, D), jnp.float32)),
)

_tc_mid = pl.pallas_call(
    _tc_mid_body,
    out_shape=jax.ShapeDtypeStruct((N, D), jnp.float32),
)

_tc_final = pl.pallas_call(
    _tc_final_body,
    out_shape=jax.ShapeDtypeStruct((N, D), jnp.float32),
)


def kernel(x, edge_index, W1, b1, W2, b2, W3, b3, W4, b4,
           g1, beta1, g2, beta2, g3, beta3):
    edge = jnp.asarray(edge_index, jnp.int32)
    src = edge[0]
    dst = edge[1]

    edge_call, deg_call = _make_sc_kernels()

    zeros_nd = jnp.zeros((N, D), jnp.float32)
    zeros_nw = jnp.zeros((N, DEG_W), jnp.float32)
    ones_cw = jnp.ones((CHUNK, DEG_W), jnp.float32)

    degp = deg_call(dst, ones_cw, zeros_nw)
    dinv, hs = _tc_first(degp, x, W1)

    b1r, b2r, b3r, b4r = (v.reshape(1, D) for v in (b1, b2, b3, b4))
    g1r, g2r, g3r = (v.reshape(1, D) for v in (g1, g2, g3))
    be1, be2, be3 = (v.reshape(1, D) for v in (beta1, beta2, beta3))

    p = edge_call(hs, src, dst, zeros_nd)
    hs = _tc_mid(p, hs, dinv, b1r, g1r, be1, W2)
    p = edge_call(hs, src, dst, zeros_nd)
    hs = _tc_mid(p, hs, dinv, b2r, g2r, be2, W3)
    p = edge_call(hs, src, dst, zeros_nd)
    hs = _tc_mid(p, hs, dinv, b3r, g3r, be3, W4)
    p = edge_call(hs, src, dst, zeros_nd)
    return _tc_final(p, hs, dinv, b4r)


# speedup vs baseline: 10.7508x; 1.0900x over previous
import functools

import jax
import jax.numpy as jnp
from jax import lax
from jax.experimental import pallas as pl
from jax.experimental.pallas import tpu as pltpu
from jax.experimental.pallas import tpu_sc as plsc

N = 10000
E = 320000
D = 128
DEG_W = 128
CHUNK = 40
NBUF = 5


def _slice_copy(src, dst, s, ns):
    rows_per_s = (N // ns // 8) * 8
    tail = N - ns * rows_per_s
    pltpu.sync_copy(src.at[pl.ds(s * rows_per_s, rows_per_s)],
                    dst.at[pl.ds(s * rows_per_s, rows_per_s)])
    if tail:
        @pl.when(s == 0)
        def _():
            pltpu.sync_copy(src.at[pl.ds(ns * rows_per_s, tail)],
                            dst.at[pl.ds(ns * rows_per_s, tail)])


def _edge_body(nc, ns, hs, srcg, dstg, zeros, out, *scratch):
    sidx = scratch[0:NBUF]
    didx = scratch[NBUF:2 * NBUF]
    rows = scratch[2 * NBUF:3 * NBUF]
    acc = scratch[3 * NBUF]
    sems = scratch[3 * NBUF + 1:]
    c = lax.axis_index("c")
    s = lax.axis_index("s")
    wid = s * nc + c
    nw = nc * ns
    per_tile = E // nw
    nchunk = per_tile // CHUNK
    base = wid * per_tile

    _slice_copy(zeros, acc, s, ns)
    plsc.subcore_barrier()

    for b in range(NBUF):
        pltpu.sync_copy(srcg.at[pl.ds(base + b * CHUNK, CHUNK)], sidx[b])
        pltpu.sync_copy(dstg.at[pl.ds(base + b * CHUNK, CHUNK)], didx[b])
        pltpu.async_copy(hs.at[sidx[b]], rows[b], sems[b])

    def outer(g, carry):
        for b in range(NBUF):
            j = g * NBUF + b
            pltpu.make_async_copy(hs.at[sidx[b]], rows[b], sems[b]).wait()
            pltpu.sync_copy(rows[b], acc.at[didx[b]], add=True)
            nj = j + NBUF

            @pl.when(nj < nchunk)
            def _():
                off = base + nj * CHUNK
                pltpu.sync_copy(srcg.at[pl.ds(off, CHUNK)], sidx[b])
                pltpu.sync_copy(dstg.at[pl.ds(off, CHUNK)], didx[b])
                pltpu.async_copy(hs.at[sidx[b]], rows[b], sems[b])
        return carry

    lax.fori_loop(0, nchunk // NBUF, outer, 0)
    plsc.subcore_barrier()
    _slice_copy(acc, out.at[c], s, ns)


def _deg_body(nc, ns, dstg, ones, zeros, out, *scratch):
    didx = scratch[0:NBUF]
    ones_v = scratch[NBUF]
    acc = scratch[NBUF + 1]
    sems = scratch[NBUF + 2:]
    c = lax.axis_index("c")
    s = lax.axis_index("s")
    wid = s * nc + c
    nw = nc * ns
    per_tile = E // nw
    nchunk = per_tile // CHUNK
    base = wid * per_tile

    pltpu.sync_copy(ones, ones_v)
    _slice_copy(zeros, acc, s, ns)
    plsc.subcore_barrier()

    for b in range(NBUF):
        pltpu.async_copy(dstg.at[pl.ds(base + b * CHUNK, CHUNK)], didx[b],
                         sems[b])

    def outer(g, carry):
        for b in range(NBUF):
            j = g * NBUF + b
            pltpu.make_async_copy(dstg.at[pl.ds(0, CHUNK)], didx[b],
                                  sems[b]).wait()
            pltpu.sync_copy(ones_v, acc.at[didx[b]], add=True)
            nj = j + NBUF

            @pl.when(nj < nchunk)
            def _():
                pltpu.async_copy(dstg.at[pl.ds(base + nj * CHUNK, CHUNK)],
                                 didx[b], sems[b])
        return carry

    lax.fori_loop(0, nchunk // NBUF, outer, 0)
    plsc.subcore_barrier()
    _slice_copy(acc, out.at[c], s, ns)


def _make_sc_kernels():
    info = plsc.get_sparse_core_info()
    nc, ns = info.num_cores, info.num_subcores
    nw = nc * ns
    nchunk = (E // nw) // CHUNK
    mesh = plsc.VectorSubcoreMesh(core_axis_name="c", subcore_axis_name="s")

    edge = pl.kernel(
        functools.partial(_edge_body, nc, ns),
        out_type=jax.ShapeDtypeStruct((nc, N, D), jnp.float32),
        mesh=mesh,
        scratch_types=(
            [pltpu.VMEM((CHUNK,), jnp.int32)] * NBUF
            + [pltpu.VMEM((CHUNK,), jnp.int32)] * NBUF
            + [pltpu.VMEM((CHUNK, D), jnp.float32)] * NBUF
            + [pltpu.VMEM_SHARED((N, D), jnp.float32)]
            + [pltpu.SemaphoreType.DMA] * NBUF
        ),
    )

    deg = pl.kernel(
        functools.partial(_deg_body, nc, ns),
        out_type=jax.ShapeDtypeStruct((nc, N, DEG_W), jnp.float32),
        mesh=mesh,
        scratch_types=(
            [pltpu.VMEM((CHUNK,), jnp.int32)] * NBUF
            + [pltpu.VMEM((CHUNK, DEG_W), jnp.float32)]
            + [pltpu.VMEM_SHARED((N, DEG_W), jnp.float32)]
            + [pltpu.SemaphoreType.DMA] * NBUF
        ),
    )
    return edge, deg, nw, nchunk


def _tc_first_body(degp_ref, x_ref, w_ref, dinv_ref, hs_ref):
    degp = degp_ref[...]
    deg = degp[0, :, 0:1] + degp[1, :, 0:1] + 1.0
    dinv = lax.rsqrt(deg)
    dinv_ref[...] = dinv
    hs_ref[...] = dinv * jnp.dot(x_ref[...], w_ref[...],
                                 preferred_element_type=jnp.float32)


def _tc_mid_body(p_ref, hs_ref, dinv_ref, b_ref, g_ref, beta_ref, w_ref, o_ref):
    p = p_ref[...]
    dinv = dinv_ref[...]
    t = dinv * (p[0] + p[1] + hs_ref[...]) + b_ref[...]
    mean = jnp.mean(t, axis=0, keepdims=True)
    var = jnp.mean((t - mean) * (t - mean), axis=0, keepdims=True)
    y = (t - mean) * lax.rsqrt(var + 1e-5) * g_ref[...] + beta_ref[...]
    r = jnp.maximum(y, 0.0)
    o_ref[...] = dinv * jnp.dot(r, w_ref[...],
                                preferred_element_type=jnp.float32)


def _tc_final_body(p_ref, hs_ref, dinv_ref, b_ref, o_ref):
    p = p_ref[...]
    o_ref[...] = dinv_ref[...] * (p[0] + p[1] + hs_ref[...]) + b_ref[...]


_tc_first = pl.pallas_call(
    _tc_first_body,
    out_shape=(jax.ShapeDtypeStruct((N, 1), jnp.float32),
               jax.ShapeDtypeStruct((N, D), jnp.float32)),
)

_tc_mid = pl.pallas_call(
    _tc_mid_body,
    out_shape=jax.ShapeDtypeStruct((N, D), jnp.float32),
)

_tc_final = pl.pallas_call(
    _tc_final_body,
    out_shape=jax.ShapeDtypeStruct((N, D), jnp.float32),
)


def kernel(x, edge_index, W1, b1, W2, b2, W3, b3, W4, b4,
           g1, beta1, g2, beta2, g3, beta3):
    edge_call, deg_call, nw, nchunk = _make_sc_kernels()

    edge = jnp.asarray(edge_index, jnp.int32)
    src = edge[0]
    dst = edge[1]

    zeros_nd = jnp.zeros((N, D), jnp.float32)
    zeros_nw = jnp.zeros((N, DEG_W), jnp.float32)
    ones_cw = jnp.ones((CHUNK, DEG_W), jnp.float32)

    degp = deg_call(dst, ones_cw, zeros_nw)
    dinv, hs = _tc_first(degp, x, W1)

    b1r, b2r, b3r, b4r = (v.reshape(1, D) for v in (b1, b2, b3, b4))
    g1r, g2r, g3r = (v.reshape(1, D) for v in (g1, g2, g3))
    be1, be2, be3 = (v.reshape(1, D) for v in (beta1, beta2, beta3))

    p = edge_call(hs, src, dst, zeros_nd)
    hs = _tc_mid(p, hs, dinv, b1r, g1r, be1, W2)
    p = edge_call(hs, src, dst, zeros_nd)
    hs = _tc_mid(p, hs, dinv, b2r, g2r, be2, W3)
    p = edge_call(hs, src, dst, zeros_nd)
    hs = _tc_mid(p, hs, dinv, b3r, g3r, be3, W4)
    p = edge_call(hs, src, dst, zeros_nd)
    return _tc_final(p, hs, dinv, b4r)


# speedup vs baseline: 26.3176x; 2.4480x over previous
import functools

import jax
import jax.numpy as jnp
from jax import lax
from jax.experimental import pallas as pl
from jax.experimental.pallas import tpu as pltpu
from jax.experimental.pallas import tpu_sc as plsc

N = 10000
E = 320000
D = 128
DEG_W = 128
CHUNK = 40
NBUF = 5


def _slice_copy(src, dst, s, ns):
    rows_per_s = (N // ns // 8) * 8
    tail = N - ns * rows_per_s
    pltpu.sync_copy(src.at[pl.ds(s * rows_per_s, rows_per_s)],
                    dst.at[pl.ds(s * rows_per_s, rows_per_s)])
    if tail:
        @pl.when(s == 0)
        def _():
            pltpu.sync_copy(src.at[pl.ds(ns * rows_per_s, tail)],
                            dst.at[pl.ds(ns * rows_per_s, tail)])


def _edge_body(nc, ns, hs, srcg, dstg, zeros, out, *scratch):
    src_all = scratch[0]
    didx = scratch[1:1 + NBUF]
    rows = scratch[1 + NBUF:1 + 2 * NBUF]
    acc = scratch[1 + 2 * NBUF]
    gsems = scratch[2 + 2 * NBUF:2 + 3 * NBUF]
    dsems = scratch[2 + 3 * NBUF:2 + 4 * NBUF]
    c = lax.axis_index("c")
    s = lax.axis_index("s")
    wid = s * nc + c
    nw = nc * ns
    per_tile = E // nw
    nchunk = per_tile // CHUNK
    base = wid * per_tile

    pltpu.sync_copy(srcg.at[pl.ds(base, per_tile)], src_all)

    _slice_copy(zeros, acc, s, ns)
    plsc.subcore_barrier()

    for b in range(NBUF):
        pltpu.async_copy(dstg.at[pl.ds(base + b * CHUNK, CHUNK)], didx[b],
                         dsems[b])
        pltpu.async_copy(hs.at[src_all.at[pl.ds(b * CHUNK, CHUNK)]], rows[b],
                         gsems[b])

    def outer(g, carry):
        for b in range(NBUF):
            j = g * NBUF + b
            pltpu.make_async_copy(hs.at[src_all.at[pl.ds(0, CHUNK)]], rows[b],
                                  gsems[b]).wait()
            pltpu.make_async_copy(dstg.at[pl.ds(0, CHUNK)], didx[b],
                                  dsems[b]).wait()
            pltpu.sync_copy(rows[b], acc.at[didx[b]], add=True)
            nj = j + NBUF

            @pl.when(nj < nchunk)
            def _():
                off = nj * CHUNK
                pltpu.async_copy(dstg.at[pl.ds(base + off, CHUNK)], didx[b],
                                 dsems[b])
                pltpu.async_copy(hs.at[src_all.at[pl.ds(off, CHUNK)]],
                                 rows[b], gsems[b])
        return carry

    lax.fori_loop(0, nchunk // NBUF, outer, 0)
    plsc.subcore_barrier()
    _slice_copy(acc, out.at[c], s, ns)


def _deg_body(nc, ns, dstg, ones, zeros, out, *scratch):
    didx = scratch[0:NBUF]
    ones_v = scratch[NBUF]
    acc = scratch[NBUF + 1]
    sems = scratch[NBUF + 2:]
    c = lax.axis_index("c")
    s = lax.axis_index("s")
    wid = s * nc + c
    nw = nc * ns
    per_tile = E // nw
    nchunk = per_tile // CHUNK
    base = wid * per_tile

    pltpu.sync_copy(ones, ones_v)
    _slice_copy(zeros, acc, s, ns)
    plsc.subcore_barrier()

    for b in range(NBUF):
        pltpu.async_copy(dstg.at[pl.ds(base + b * CHUNK, CHUNK)], didx[b],
                         sems[b])

    def outer(g, carry):
        for b in range(NBUF):
            j = g * NBUF + b
            pltpu.make_async_copy(dstg.at[pl.ds(0, CHUNK)], didx[b],
                                  sems[b]).wait()
            pltpu.sync_copy(ones_v, acc.at[didx[b]], add=True)
            nj = j + NBUF

            @pl.when(nj < nchunk)
            def _():
                pltpu.async_copy(dstg.at[pl.ds(base + nj * CHUNK, CHUNK)],
                                 didx[b], sems[b])
        return carry

    lax.fori_loop(0, nchunk // NBUF, outer, 0)
    plsc.subcore_barrier()
    _slice_copy(acc, out.at[c], s, ns)


def _make_sc_kernels():
    info = plsc.get_sparse_core_info()
    nc, ns = info.num_cores, info.num_subcores
    nw = nc * ns
    nchunk = (E // nw) // CHUNK
    mesh = plsc.VectorSubcoreMesh(core_axis_name="c", subcore_axis_name="s")

    per_tile = E // nw
    edge = pl.kernel(
        functools.partial(_edge_body, nc, ns),
        out_type=jax.ShapeDtypeStruct((nc, N, D), jnp.float32),
        mesh=mesh,
        scratch_types=(
            [pltpu.VMEM((per_tile,), jnp.int32)]
            + [pltpu.VMEM((CHUNK,), jnp.int32)] * NBUF
            + [pltpu.VMEM((CHUNK, D), jnp.float32)] * NBUF
            + [pltpu.VMEM_SHARED((N, D), jnp.float32)]
            + [pltpu.SemaphoreType.DMA] * NBUF
            + [pltpu.SemaphoreType.DMA] * NBUF
        ),
    )

    deg = pl.kernel(
        functools.partial(_deg_body, nc, ns),
        out_type=jax.ShapeDtypeStruct((nc, N, DEG_W), jnp.float32),
        mesh=mesh,
        scratch_types=(
            [pltpu.VMEM((CHUNK,), jnp.int32)] * NBUF
            + [pltpu.VMEM((CHUNK, DEG_W), jnp.float32)]
            + [pltpu.VMEM_SHARED((N, DEG_W), jnp.float32)]
            + [pltpu.SemaphoreType.DMA] * NBUF
        ),
    )
    return edge, deg, nw, nchunk


def _tc_first_body(degp_ref, x_ref, w_ref, dinv_ref, hs_ref):
    degp = degp_ref[...]
    deg = degp[0, :, 0:1] + degp[1, :, 0:1] + 1.0
    dinv = lax.rsqrt(deg)
    dinv_ref[...] = dinv
    hs_ref[...] = dinv * jnp.dot(x_ref[...], w_ref[...],
                                 preferred_element_type=jnp.float32)


def _tc_mid_body(p_ref, hs_ref, dinv_ref, b_ref, g_ref, beta_ref, w_ref, o_ref):
    p = p_ref[...]
    dinv = dinv_ref[...]
    t = dinv * (p[0] + p[1] + hs_ref[...]) + b_ref[...]
    mean = jnp.mean(t, axis=0, keepdims=True)
    var = jnp.mean((t - mean) * (t - mean), axis=0, keepdims=True)
    y = (t - mean) * lax.rsqrt(var + 1e-5) * g_ref[...] + beta_ref[...]
    r = jnp.maximum(y, 0.0)
    o_ref[...] = dinv * jnp.dot(r, w_ref[...],
                                preferred_element_type=jnp.float32)


def _tc_final_body(p_ref, hs_ref, dinv_ref, b_ref, o_ref):
    p = p_ref[...]
    o_ref[...] = dinv_ref[...] * (p[0] + p[1] + hs_ref[...]) + b_ref[...]


_tc_first = pl.pallas_call(
    _tc_first_body,
    out_shape=(jax.ShapeDtypeStruct((N, 1), jnp.float32),
               jax.ShapeDtypeStruct((N, D), jnp.float32)),
)

_tc_mid = pl.pallas_call(
    _tc_mid_body,
    out_shape=jax.ShapeDtypeStruct((N, D), jnp.float32),
)

_tc_final = pl.pallas_call(
    _tc_final_body,
    out_shape=jax.ShapeDtypeStruct((N, D), jnp.float32),
)


def kernel(x, edge_index, W1, b1, W2, b2, W3, b3, W4, b4,
           g1, beta1, g2, beta2, g3, beta3):
    edge_call, deg_call, nw, nchunk = _make_sc_kernels()

    edge = jnp.asarray(edge_index, jnp.int32)
    src = edge[0]
    dst = edge[1]

    zeros_nd = jnp.zeros((N, D), jnp.float32)
    zeros_nw = jnp.zeros((N, DEG_W), jnp.float32)
    ones_cw = jnp.ones((CHUNK, DEG_W), jnp.float32)

    degp = deg_call(dst, ones_cw, zeros_nw)
    dinv, hs = _tc_first(degp, x, W1)

    b1r, b2r, b3r, b4r = (v.reshape(1, D) for v in (b1, b2, b3, b4))
    g1r, g2r, g3r = (v.reshape(1, D) for v in (g1, g2, g3))
    be1, be2, be3 = (v.reshape(1, D) for v in (beta1, beta2, beta3))

    p = edge_call(hs, src, dst, zeros_nd)
    hs = _tc_mid(p, hs, dinv, b1r, g1r, be1, W2)
    p = edge_call(hs, src, dst, zeros_nd)
    hs = _tc_mid(p, hs, dinv, b2r, g2r, be2, W3)
    p = edge_call(hs, src, dst, zeros_nd)
    hs = _tc_mid(p, hs, dinv, b3r, g3r, be3, W4)
    p = edge_call(hs, src, dst, zeros_nd)
    return _tc_final(p, hs, dinv, b4r)


# speedup vs baseline: 28.9275x; 1.0992x over previous
import functools

import jax
import jax.numpy as jnp
from jax import lax
from jax.experimental import pallas as pl
from jax.experimental.pallas import tpu as pltpu
from jax.experimental.pallas import tpu_sc as plsc

N = 10000
E = 320000
D = 128
CHUNK = 40
NBUF = 5


def _slice_copy(src, dst, s, ns):
    rows_per_s = (N // ns // 8) * 8
    tail = N - ns * rows_per_s
    pltpu.sync_copy(src.at[pl.ds(s * rows_per_s, rows_per_s)],
                    dst.at[pl.ds(s * rows_per_s, rows_per_s)])
    if tail:
        @pl.when(s == 0)
        def _():
            pltpu.sync_copy(src.at[pl.ds(ns * rows_per_s, tail)],
                            dst.at[pl.ds(ns * rows_per_s, tail)])


def _edge_body(nc, ns, hs, srcg, dstg, zeros, out, *scratch):
    src_all = scratch[0]
    didx = scratch[1:1 + NBUF]
    rows = scratch[1 + NBUF:1 + 2 * NBUF]
    acc = scratch[1 + 2 * NBUF]
    gsems = scratch[2 + 2 * NBUF:2 + 3 * NBUF]
    dsems = scratch[2 + 3 * NBUF:2 + 4 * NBUF]
    c = lax.axis_index("c")
    s = lax.axis_index("s")
    wid = s * nc + c
    nw = nc * ns
    per_tile = E // nw
    nchunk = per_tile // CHUNK
    base = wid * per_tile

    pltpu.sync_copy(srcg.at[pl.ds(base, per_tile)], src_all)

    _slice_copy(zeros, acc, s, ns)
    plsc.subcore_barrier()

    for b in range(NBUF):
        pltpu.async_copy(dstg.at[pl.ds(base + b * CHUNK, CHUNK)], didx[b],
                         dsems[b])
        pltpu.async_copy(hs.at[src_all.at[pl.ds(b * CHUNK, CHUNK)]], rows[b],
                         gsems[b])

    def outer(g, carry):
        for b in range(NBUF):
            j = g * NBUF + b
            pltpu.make_async_copy(hs.at[src_all.at[pl.ds(0, CHUNK)]], rows[b],
                                  gsems[b]).wait()
            pltpu.make_async_copy(dstg.at[pl.ds(0, CHUNK)], didx[b],
                                  dsems[b]).wait()
            pltpu.sync_copy(rows[b], acc.at[didx[b]], add=True)
            nj = j + NBUF

            @pl.when(nj < nchunk)
            def _():
                off = nj * CHUNK
                pltpu.async_copy(dstg.at[pl.ds(base + off, CHUNK)], didx[b],
                                 dsems[b])
                pltpu.async_copy(hs.at[src_all.at[pl.ds(off, CHUNK)]],
                                 rows[b], gsems[b])
        return carry

    lax.fori_loop(0, nchunk // NBUF, outer, 0)
    plsc.subcore_barrier()
    _slice_copy(acc, out.at[c], s, ns)


def _deg_body(nc, ns, dstg, ones, zeros, out, *scratch):
    didx = scratch[0:NBUF]
    ones_v = scratch[NBUF]
    acc = scratch[NBUF + 1]
    sems = scratch[NBUF + 2:]
    c = lax.axis_index("c")
    s = lax.axis_index("s")
    wid = s * nc + c
    nw = nc * ns
    per_tile = E // nw
    nchunk = per_tile // CHUNK
    base = wid * per_tile

    pltpu.sync_copy(ones, ones_v)
    @pl.when(s == 0)
    def _():
        pltpu.sync_copy(zeros, acc)
    plsc.subcore_barrier()

    for b in range(NBUF):
        pltpu.async_copy(dstg.at[pl.ds(base + b * CHUNK, CHUNK)], didx[b],
                         sems[b])

    def outer(g, carry):
        for b in range(NBUF):
            j = g * NBUF + b
            pltpu.make_async_copy(dstg.at[pl.ds(0, CHUNK)], didx[b],
                                  sems[b]).wait()
            pltpu.sync_copy(ones_v, acc.at[didx[b]], add=True)
            nj = j + NBUF

            @pl.when(nj < nchunk)
            def _():
                pltpu.async_copy(dstg.at[pl.ds(base + nj * CHUNK, CHUNK)],
                                 didx[b], sems[b])
        return carry

    lax.fori_loop(0, nchunk // NBUF, outer, 0)
    plsc.subcore_barrier()

    @pl.when(s == 0)
    def _():
        pltpu.sync_copy(acc, out.at[c])


def _make_sc_kernels():
    info = plsc.get_sparse_core_info()
    nc, ns = info.num_cores, info.num_subcores
    nw = nc * ns
    nchunk = (E // nw) // CHUNK
    mesh = plsc.VectorSubcoreMesh(core_axis_name="c", subcore_axis_name="s")

    per_tile = E // nw
    edge = pl.kernel(
        functools.partial(_edge_body, nc, ns),
        out_type=jax.ShapeDtypeStruct((nc, N, D), jnp.float32),
        mesh=mesh,
        scratch_types=(
            [pltpu.VMEM((per_tile,), jnp.int32)]
            + [pltpu.VMEM((CHUNK,), jnp.int32)] * NBUF
            + [pltpu.VMEM((CHUNK, D), jnp.float32)] * NBUF
            + [pltpu.VMEM_SHARED((N, D), jnp.float32)]
            + [pltpu.SemaphoreType.DMA] * NBUF
            + [pltpu.SemaphoreType.DMA] * NBUF
        ),
    )

    deg = pl.kernel(
        functools.partial(_deg_body, nc, ns),
        out_type=jax.ShapeDtypeStruct((nc, N), jnp.float32),
        mesh=mesh,
        scratch_types=(
            [pltpu.VMEM((CHUNK,), jnp.int32)] * NBUF
            + [pltpu.VMEM((CHUNK,), jnp.float32)]
            + [pltpu.VMEM_SHARED((N,), jnp.float32)]
            + [pltpu.SemaphoreType.DMA] * NBUF
        ),
    )
    return edge, deg, nw, nchunk


def _tc_first_body(degp_ref, x_ref, w_ref, dinv_ref, hs_ref):
    degp = degp_ref[...]
    deg = degp[:, 0:1] + degp[:, 1:2] + 1.0
    dinv = lax.rsqrt(deg)
    dinv_ref[...] = dinv
    hs_ref[...] = dinv * jnp.dot(x_ref[...], w_ref[...],
                                 preferred_element_type=jnp.float32)


def _tc_mid_body(p_ref, hs_ref, dinv_ref, b_ref, g_ref, beta_ref, w_ref, o_ref):
    p = p_ref[...]
    dinv = dinv_ref[...]
    t = dinv * (p[0] + p[1] + hs_ref[...]) + b_ref[...]
    mean = jnp.mean(t, axis=0, keepdims=True)
    var = jnp.mean((t - mean) * (t - mean), axis=0, keepdims=True)
    y = (t - mean) * lax.rsqrt(var + 1e-5) * g_ref[...] + beta_ref[...]
    r = jnp.maximum(y, 0.0)
    o_ref[...] = dinv * jnp.dot(r, w_ref[...],
                                preferred_element_type=jnp.float32)


def _tc_final_body(p_ref, hs_ref, dinv_ref, b_ref, o_ref):
    p = p_ref[...]
    o_ref[...] = dinv_ref[...] * (p[0] + p[1] + hs_ref[...]) + b_ref[...]


_tc_first = pl.pallas_call(
    _tc_first_body,
    out_shape=(jax.ShapeDtypeStruct((N, 1), jnp.float32),
               jax.ShapeDtypeStruct((N, D), jnp.float32)),
)

_tc_mid = pl.pallas_call(
    _tc_mid_body,
    out_shape=jax.ShapeDtypeStruct((N, D), jnp.float32),
)

_tc_final = pl.pallas_call(
    _tc_final_body,
    out_shape=jax.ShapeDtypeStruct((N, D), jnp.float32),
)


def kernel(x, edge_index, W1, b1, W2, b2, W3, b3, W4, b4,
           g1, beta1, g2, beta2, g3, beta3):
    edge_call, deg_call, nw, nchunk = _make_sc_kernels()

    edge = jnp.asarray(edge_index, jnp.int32)
    src = edge[0]
    dst = edge[1]

    zeros_nd = jnp.zeros((N, D), jnp.float32)
    zeros_n = jnp.zeros((N,), jnp.float32)
    ones_c = jnp.ones((CHUNK,), jnp.float32)

    degp = deg_call(dst, ones_c, zeros_n)
    dinv, hs = _tc_first(jnp.transpose(degp), x, W1)

    b1r, b2r, b3r, b4r = (v.reshape(1, D) for v in (b1, b2, b3, b4))
    g1r, g2r, g3r = (v.reshape(1, D) for v in (g1, g2, g3))
    be1, be2, be3 = (v.reshape(1, D) for v in (beta1, beta2, beta3))

    p = edge_call(hs, src, dst, zeros_nd)
    hs = _tc_mid(p, hs, dinv, b1r, g1r, be1, W2)
    p = edge_call(hs, src, dst, zeros_nd)
    hs = _tc_mid(p, hs, dinv, b2r, g2r, be2, W3)
    p = edge_call(hs, src, dst, zeros_nd)
    hs = _tc_mid(p, hs, dinv, b3r, g3r, be3, W4)
    p = edge_call(hs, src, dst, zeros_nd)
    return _tc_final(p, hs, dinv, b4r)


# speedup vs baseline: 29.5569x; 1.0218x over previous
import functools

import jax
import jax.numpy as jnp
from jax import lax
from jax.experimental import pallas as pl
from jax.experimental.pallas import tpu as pltpu
from jax.experimental.pallas import tpu_sc as plsc

N = 10000
E = 320000
D = 128
CHUNK = 40
DCHUNK = 80
NBUF = 5


def _slice_copy(src, dst, s, ns):
    rows_per_s = (N // ns // 8) * 8
    tail = N - ns * rows_per_s
    pltpu.sync_copy(src.at[pl.ds(s * rows_per_s, rows_per_s)],
                    dst.at[pl.ds(s * rows_per_s, rows_per_s)])
    if tail:
        @pl.when(s == 0)
        def _():
            pltpu.sync_copy(src.at[pl.ds(ns * rows_per_s, tail)],
                            dst.at[pl.ds(ns * rows_per_s, tail)])


def _edge_body(nc, ns, hs, srcg, dstg, zeros, out, *scratch):
    src_all = scratch[0]
    didx = scratch[1:1 + NBUF]
    rows = scratch[1 + NBUF:1 + 2 * NBUF]
    acc = scratch[1 + 2 * NBUF]
    gsems = scratch[2 + 2 * NBUF:2 + 3 * NBUF]
    dsems = scratch[2 + 3 * NBUF:2 + 4 * NBUF]
    c = lax.axis_index("c")
    s = lax.axis_index("s")
    wid = s * nc + c
    nw = nc * ns
    per_tile = E // nw
    nchunk = per_tile // CHUNK
    base = wid * per_tile

    pltpu.sync_copy(srcg.at[pl.ds(base, per_tile)], src_all)

    _slice_copy(zeros, acc, s, ns)
    plsc.subcore_barrier()

    for b in range(NBUF):
        pltpu.async_copy(dstg.at[pl.ds(base + b * CHUNK, CHUNK)], didx[b],
                         dsems[b])
        pltpu.async_copy(hs.at[src_all.at[pl.ds(b * CHUNK, CHUNK)]], rows[b],
                         gsems[b])

    def outer(g, carry):
        for b in range(NBUF):
            j = g * NBUF + b
            pltpu.make_async_copy(hs.at[src_all.at[pl.ds(0, CHUNK)]], rows[b],
                                  gsems[b]).wait()
            pltpu.make_async_copy(dstg.at[pl.ds(0, CHUNK)], didx[b],
                                  dsems[b]).wait()
            pltpu.sync_copy(rows[b], acc.at[didx[b]], add=True)
            nj = j + NBUF

            @pl.when(nj < nchunk)
            def _():
                off = nj * CHUNK
                pltpu.async_copy(dstg.at[pl.ds(base + off, CHUNK)], didx[b],
                                 dsems[b])
                pltpu.async_copy(hs.at[src_all.at[pl.ds(off, CHUNK)]],
                                 rows[b], gsems[b])
        return carry

    lax.fori_loop(0, nchunk // NBUF, outer, 0)
    plsc.subcore_barrier()
    _slice_copy(acc, out.at[c], s, ns)


def _deg_body(nc, ns, dstg, ones, zeros, out, *scratch):
    didx = scratch[0:NBUF]
    ones_v = scratch[NBUF]
    acc = scratch[NBUF + 1]
    sems = scratch[NBUF + 2:]
    c = lax.axis_index("c")
    s = lax.axis_index("s")
    wid = s * nc + c
    nw = nc * ns
    per_tile = E // nw
    nchunk = per_tile // DCHUNK
    base = wid * per_tile

    pltpu.sync_copy(ones, ones_v)
    @pl.when(s == 0)
    def _():
        pltpu.sync_copy(zeros, acc)
    plsc.subcore_barrier()

    for b in range(NBUF):
        pltpu.async_copy(dstg.at[pl.ds(base + b * DCHUNK, DCHUNK)], didx[b],
                         sems[b])

    def outer(g, carry):
        for b in range(NBUF):
            j = g * NBUF + b
            pltpu.make_async_copy(dstg.at[pl.ds(0, DCHUNK)], didx[b],
                                  sems[b]).wait()
            pltpu.sync_copy(ones_v, acc.at[didx[b]], add=True)
            nj = j + NBUF

            @pl.when(nj < nchunk)
            def _():
                pltpu.async_copy(dstg.at[pl.ds(base + nj * DCHUNK, DCHUNK)],
                                 didx[b], sems[b])
        return carry

    lax.fori_loop(0, nchunk // NBUF, outer, 0)
    plsc.subcore_barrier()

    @pl.when(s == 0)
    def _():
        pltpu.sync_copy(acc, out.at[c])


def _make_sc_kernels():
    info = plsc.get_sparse_core_info()
    nc, ns = info.num_cores, info.num_subcores
    nw = nc * ns
    nchunk = (E // nw) // CHUNK
    mesh = plsc.VectorSubcoreMesh(core_axis_name="c", subcore_axis_name="s")

    per_tile = E // nw
    edge = pl.kernel(
        functools.partial(_edge_body, nc, ns),
        out_type=jax.ShapeDtypeStruct((nc, N, D), jnp.float32),
        mesh=mesh,
        scratch_types=(
            [pltpu.VMEM((per_tile,), jnp.int32)]
            + [pltpu.VMEM((CHUNK,), jnp.int32)] * NBUF
            + [pltpu.VMEM((CHUNK, D), jnp.float32)] * NBUF
            + [pltpu.VMEM_SHARED((N, D), jnp.float32)]
            + [pltpu.SemaphoreType.DMA] * NBUF
            + [pltpu.SemaphoreType.DMA] * NBUF
        ),
    )

    deg = pl.kernel(
        functools.partial(_deg_body, nc, ns),
        out_type=jax.ShapeDtypeStruct((nc, N), jnp.float32),
        mesh=mesh,
        scratch_types=(
            [pltpu.VMEM((DCHUNK,), jnp.int32)] * NBUF
            + [pltpu.VMEM((DCHUNK,), jnp.float32)]
            + [pltpu.VMEM_SHARED((N,), jnp.float32)]
            + [pltpu.SemaphoreType.DMA] * NBUF
        ),
    )
    return edge, deg, nw, nchunk


def _tc_mm_body(x_ref, w_ref, h_ref):
    h_ref[...] = jnp.dot(x_ref[...], w_ref[...],
                         preferred_element_type=jnp.float32)


def _tc_first_body(degp_ref, h_ref, dinv_ref, hs_ref):
    degp = degp_ref[...]
    deg = degp[:, 0:1] + degp[:, 1:2] + 1.0
    dinv = lax.rsqrt(deg)
    dinv_ref[...] = dinv
    hs_ref[...] = dinv * h_ref[...]


def _tc_mid_body(p_ref, hs_ref, dinv_ref, b_ref, g_ref, beta_ref, w_ref, o_ref):
    p = p_ref[...]
    dinv = dinv_ref[...]
    t = dinv * (p[0] + p[1] + hs_ref[...]) + b_ref[...]
    mean = jnp.mean(t, axis=0, keepdims=True)
    var = jnp.mean((t - mean) * (t - mean), axis=0, keepdims=True)
    y = (t - mean) * lax.rsqrt(var + 1e-5) * g_ref[...] + beta_ref[...]
    r = jnp.maximum(y, 0.0)
    o_ref[...] = dinv * jnp.dot(r, w_ref[...],
                                preferred_element_type=jnp.float32)


def _tc_final_body(p_ref, hs_ref, dinv_ref, b_ref, o_ref):
    p = p_ref[...]
    o_ref[...] = dinv_ref[...] * (p[0] + p[1] + hs_ref[...]) + b_ref[...]


_tc_mm = pl.pallas_call(
    _tc_mm_body,
    out_shape=jax.ShapeDtypeStruct((N, D), jnp.float32),
)

_tc_first = pl.pallas_call(
    _tc_first_body,
    out_shape=(jax.ShapeDtypeStruct((N, 1), jnp.float32),
               jax.ShapeDtypeStruct((N, D), jnp.float32)),
)

_tc_mid = pl.pallas_call(
    _tc_mid_body,
    out_shape=jax.ShapeDtypeStruct((N, D), jnp.float32),
)

_tc_final = pl.pallas_call(
    _tc_final_body,
    out_shape=jax.ShapeDtypeStruct((N, D), jnp.float32),
)


def kernel(x, edge_index, W1, b1, W2, b2, W3, b3, W4, b4,
           g1, beta1, g2, beta2, g3, beta3):
    edge_call, deg_call, nw, nchunk = _make_sc_kernels()

    edge = jnp.asarray(edge_index, jnp.int32)
    src = edge[0]
    dst = edge[1]

    zeros_nd = jnp.zeros((N, D), jnp.float32)
    zeros_n = jnp.zeros((N,), jnp.float32)
    ones_c = jnp.ones((DCHUNK,), jnp.float32)

    h = _tc_mm(x, W1)
    degp = deg_call(dst, ones_c, zeros_n)
    dinv, hs = _tc_first(jnp.transpose(degp), h)

    b1r, b2r, b3r, b4r = (v.reshape(1, D) for v in (b1, b2, b3, b4))
    g1r, g2r, g3r = (v.reshape(1, D) for v in (g1, g2, g3))
    be1, be2, be3 = (v.reshape(1, D) for v in (beta1, beta2, beta3))

    p = edge_call(hs, src, dst, zeros_nd)
    hs = _tc_mid(p, hs, dinv, b1r, g1r, be1, W2)
    p = edge_call(hs, src, dst, zeros_nd)
    hs = _tc_mid(p, hs, dinv, b2r, g2r, be2, W3)
    p = edge_call(hs, src, dst, zeros_nd)
    hs = _tc_mid(p, hs, dinv, b3r, g3r, be3, W4)
    p = edge_call(hs, src, dst, zeros_nd)
    return _tc_final(p, hs, dinv, b4r)


# speedup vs baseline: 30.1849x; 1.0212x over previous
import functools

import jax
import jax.numpy as jnp
from jax import lax
from jax.experimental import pallas as pl
from jax.experimental.pallas import tpu as pltpu
from jax.experimental.pallas import tpu_sc as plsc

N = 10000
E = 320000
D = 128
CHUNK = 40
DCHUNK = 80
NBUF = 5


def _slice_copy(src, dst, s, ns):
    rows_per_s = (N // ns // 8) * 8
    tail = N - ns * rows_per_s
    pltpu.sync_copy(src.at[pl.ds(s * rows_per_s, rows_per_s)],
                    dst.at[pl.ds(s * rows_per_s, rows_per_s)])
    if tail:
        @pl.when(s == 0)
        def _():
            pltpu.sync_copy(src.at[pl.ds(ns * rows_per_s, tail)],
                            dst.at[pl.ds(ns * rows_per_s, tail)])


def _edge_body(nc, ns, hs, srcg, dstg, zeros, out, *scratch):
    src_all = scratch[0]
    didx = scratch[1:1 + NBUF]
    rows = scratch[1 + NBUF:1 + 2 * NBUF]
    acc = scratch[1 + 2 * NBUF]
    gsems = scratch[2 + 2 * NBUF:2 + 3 * NBUF]
    dsems = scratch[2 + 3 * NBUF:2 + 4 * NBUF]
    c = lax.axis_index("c")
    s = lax.axis_index("s")
    wid = s * nc + c
    nw = nc * ns
    per_tile = E // nw
    nchunk = per_tile // CHUNK
    base = wid * per_tile

    pltpu.sync_copy(srcg.at[pl.ds(base, per_tile)], src_all)

    for b in range(NBUF):
        pltpu.async_copy(dstg.at[pl.ds(base + b * CHUNK, CHUNK)], didx[b],
                         dsems[b])
        pltpu.async_copy(hs.at[src_all.at[pl.ds(b * CHUNK, CHUNK)]], rows[b],
                         gsems[b])

    @pl.when(c == 0)
    def _():
        _slice_copy(hs, acc, s, ns)

    @pl.when(c != 0)
    def _():
        _slice_copy(zeros, acc, s, ns)

    plsc.subcore_barrier()

    def outer(g, carry):
        for b in range(NBUF):
            j = g * NBUF + b
            pltpu.make_async_copy(hs.at[src_all.at[pl.ds(0, CHUNK)]], rows[b],
                                  gsems[b]).wait()
            pltpu.make_async_copy(dstg.at[pl.ds(0, CHUNK)], didx[b],
                                  dsems[b]).wait()
            pltpu.sync_copy(rows[b], acc.at[didx[b]], add=True)
            nj = j + NBUF

            @pl.when(nj < nchunk)
            def _():
                off = nj * CHUNK
                pltpu.async_copy(dstg.at[pl.ds(base + off, CHUNK)], didx[b],
                                 dsems[b])
                pltpu.async_copy(hs.at[src_all.at[pl.ds(off, CHUNK)]],
                                 rows[b], gsems[b])
        return carry

    lax.fori_loop(0, nchunk // NBUF, outer, 0)
    plsc.subcore_barrier()
    _slice_copy(acc, out.at[c], s, ns)


def _deg_body(nc, ns, dstg, ones, zeros, out, *scratch):
    didx = scratch[0:NBUF]
    ones_v = scratch[NBUF]
    acc = scratch[NBUF + 1]
    sems = scratch[NBUF + 2:]
    c = lax.axis_index("c")
    s = lax.axis_index("s")
    wid = s * nc + c
    nw = nc * ns
    per_tile = E // nw
    nchunk = per_tile // DCHUNK
    base = wid * per_tile

    pltpu.sync_copy(ones, ones_v)
    @pl.when(s == 0)
    def _():
        pltpu.sync_copy(zeros, acc)
    plsc.subcore_barrier()

    for b in range(NBUF):
        pltpu.async_copy(dstg.at[pl.ds(base + b * DCHUNK, DCHUNK)], didx[b],
                         sems[b])

    def outer(g, carry):
        for b in range(NBUF):
            j = g * NBUF + b
            pltpu.make_async_copy(dstg.at[pl.ds(0, DCHUNK)], didx[b],
                                  sems[b]).wait()
            pltpu.sync_copy(ones_v, acc.at[didx[b]], add=True)
            nj = j + NBUF

            @pl.when(nj < nchunk)
            def _():
                pltpu.async_copy(dstg.at[pl.ds(base + nj * DCHUNK, DCHUNK)],
                                 didx[b], sems[b])
        return carry

    lax.fori_loop(0, nchunk // NBUF, outer, 0)
    plsc.subcore_barrier()

    @pl.when(s == 0)
    def _():
        pltpu.sync_copy(acc, out.at[c])


def _make_sc_kernels():
    info = plsc.get_sparse_core_info()
    nc, ns = info.num_cores, info.num_subcores
    nw = nc * ns
    nchunk = (E // nw) // CHUNK
    mesh = plsc.VectorSubcoreMesh(core_axis_name="c", subcore_axis_name="s")

    per_tile = E // nw
    edge = pl.kernel(
        functools.partial(_edge_body, nc, ns),
        out_type=jax.ShapeDtypeStruct((nc, N, D), jnp.float32),
        mesh=mesh,
        scratch_types=(
            [pltpu.VMEM((per_tile,), jnp.int32)]
            + [pltpu.VMEM((CHUNK,), jnp.int32)] * NBUF
            + [pltpu.VMEM((CHUNK, D), jnp.float32)] * NBUF
            + [pltpu.VMEM_SHARED((N, D), jnp.float32)]
            + [pltpu.SemaphoreType.DMA] * NBUF
            + [pltpu.SemaphoreType.DMA] * NBUF
        ),
    )

    deg = pl.kernel(
        functools.partial(_deg_body, nc, ns),
        out_type=jax.ShapeDtypeStruct((nc, N), jnp.float32),
        mesh=mesh,
        scratch_types=(
            [pltpu.VMEM((DCHUNK,), jnp.int32)] * NBUF
            + [pltpu.VMEM((DCHUNK,), jnp.float32)]
            + [pltpu.VMEM_SHARED((N,), jnp.float32)]
            + [pltpu.SemaphoreType.DMA] * NBUF
        ),
    )
    return edge, deg, nw, nchunk


def _tc_mm_body(x_ref, w_ref, h_ref):
    h_ref[...] = jnp.dot(x_ref[...], w_ref[...],
                         preferred_element_type=jnp.float32)


def _tc_first_body(degp_ref, h_ref, dinv_ref, hs_ref):
    degp = degp_ref[...]
    deg = degp[:, 0:1] + degp[:, 1:2] + 1.0
    dinv = lax.rsqrt(deg)
    dinv_ref[...] = dinv
    hs_ref[...] = dinv * h_ref[...]


def _tc_mid_body(p_ref, dinv_ref, b_ref, g_ref, beta_ref, w_ref, o_ref):
    p = p_ref[...]
    dinv = dinv_ref[...]
    t = dinv * (p[0] + p[1]) + b_ref[...]
    mean = jnp.mean(t, axis=0, keepdims=True)
    var = jnp.mean((t - mean) * (t - mean), axis=0, keepdims=True)
    y = (t - mean) * lax.rsqrt(var + 1e-5) * g_ref[...] + beta_ref[...]
    r = jnp.maximum(y, 0.0)
    o_ref[...] = dinv * jnp.dot(r, w_ref[...],
                                preferred_element_type=jnp.float32)


def _tc_final_body(p_ref, dinv_ref, b_ref, o_ref):
    p = p_ref[...]
    o_ref[...] = dinv_ref[...] * (p[0] + p[1]) + b_ref[...]


_tc_mm = pl.pallas_call(
    _tc_mm_body,
    out_shape=jax.ShapeDtypeStruct((N, D), jnp.float32),
)

_tc_first = pl.pallas_call(
    _tc_first_body,
    out_shape=(jax.ShapeDtypeStruct((N, 1), jnp.float32),
               jax.ShapeDtypeStruct((N, D), jnp.float32)),
)

_tc_mid = pl.pallas_call(
    _tc_mid_body,
    out_shape=jax.ShapeDtypeStruct((N, D), jnp.float32),
)

_tc_final = pl.pallas_call(
    _tc_final_body,
    out_shape=jax.ShapeDtypeStruct((N, D), jnp.float32),
)


def kernel(x, edge_index, W1, b1, W2, b2, W3, b3, W4, b4,
           g1, beta1, g2, beta2, g3, beta3):
    edge_call, deg_call, nw, nchunk = _make_sc_kernels()

    edge = jnp.asarray(edge_index, jnp.int32)
    src = edge[0]
    dst = edge[1]

    zeros_nd = jnp.zeros((N, D), jnp.float32)
    zeros_n = jnp.zeros((N,), jnp.float32)
    ones_c = jnp.ones((DCHUNK,), jnp.float32)

    h = _tc_mm(x, W1)
    degp = deg_call(dst, ones_c, zeros_n)
    dinv, hs = _tc_first(jnp.transpose(degp), h)

    b1r, b2r, b3r, b4r = (v.reshape(1, D) for v in (b1, b2, b3, b4))
    g1r, g2r, g3r = (v.reshape(1, D) for v in (g1, g2, g3))
    be1, be2, be3 = (v.reshape(1, D) for v in (beta1, beta2, beta3))

    p = edge_call(hs, src, dst, zeros_nd)
    hs = _tc_mid(p, dinv, b1r, g1r, be1, W2)
    p = edge_call(hs, src, dst, zeros_nd)
    hs = _tc_mid(p, dinv, b2r, g2r, be2, W3)
    p = edge_call(hs, src, dst, zeros_nd)
    hs = _tc_mid(p, dinv, b3r, g3r, be3, W4)
    p = edge_call(hs, src, dst, zeros_nd)
    return _tc_final(p, dinv, b4r)
